# baseline (device time: 28263 ns/iter reference)
import jax
import jax.numpy as jnp
from jax import lax
from jax.experimental import pallas as pl
from jax.experimental.pallas import tpu as pltpu

N_DEV = 4
G = 4


def _gelu(z):
    return 0.5 * z * (1.0 + jnp.tanh(0.7978845608 * (z + 0.044715 * z * z * z)))


def kernel(A, B):
    m, k = A.shape
    _, n = B.shape
    mh, nh = m // 2, n // 2
    mq = m // 4
    cw = nh // G
    f32, bf16 = jnp.float32, jnp.bfloat16

    def body(
        a_ref, b_ref, out_ref,
        s1a_ref, s1b_ref, zka_ref, zkb_ref, r1a_ref, r1b_ref,
        sua_ref, sub_ref, r2a_ref, r2b_ref,
        ga_ref, gb_ref, r3a_ref, r3b_ref,
        r4a_ref, r4b_ref,
        send_sems, recv_sems,
    ):
        my = lax.axis_index("i")
        mx = my // 2
        myy = (my ^ (my >> 1)) & 1
        xn = my ^ 3
        yn = my ^ 1

        rx_me = mh * mx
        rx_nb = mh * (1 - mx)
        ry_me = mh * myy
        ry_nb = mh * (1 - myy)
        o_sa = mq * (1 - myy)
        o_ka = mq * myy
        o_sb = mq * (1 - mx)
        o_kb = mq * mx
        ca = [slice(g * cw, (g + 1) * cw) for g in range(G)]
        cb = [slice(nh + g * cw, nh + (g + 1) * cw) for g in range(G)]

        barrier = pltpu.get_barrier_semaphore()
        for nbr in (xn, yn):
            pl.semaphore_signal(
                barrier, inc=1,
                device_id=(nbr,), device_id_type=pl.DeviceIdType.MESH,
            )
        pl.semaphore_wait(barrier, 2)

        def exchange(src, dst, ph, path, g, peer):
            return pltpu.make_async_remote_copy(
                src_ref=src, dst_ref=dst,
                send_sem=send_sems.at[ph, path, g],
                recv_sem=recv_sems.at[ph, path, g],
                device_id=(peer,), device_id_type=pl.DeviceIdType.MESH,
            )

        a_xnb = a_ref[pl.ds(rx_nb, mh), :].astype(bf16)
        a_ynb = a_ref[pl.ds(ry_nb, mh), :].astype(bf16)
        a_xme = a_ref[pl.ds(rx_me, mh), :].astype(bf16)
        a_yme = a_ref[pl.ds(ry_me, mh), :].astype(bf16)
        b_bf = b_ref[...].astype(bf16)

        p1 = {}
        for g in range(G):
            s1a_ref[g] = jnp.dot(
                a_xnb, b_bf[:, ca[g]], preferred_element_type=f32
            ).astype(bf16)
            p1["a", g] = exchange(
                s1a_ref.at[g], r1a_ref.at[g], 0, 0, g, xn
            )
            p1["a", g].start()
            s1b_ref[g] = jnp.dot(
                a_ynb, b_bf[:, cb[g]], preferred_element_type=f32
            ).astype(bf16)
            p1["b", g] = exchange(
                s1b_ref.at[g], r1b_ref.at[g], 0, 1, g, yn
            )
            p1["b", g].start()
        for g in range(G):
            zka_ref[g] = jnp.dot(
                a_xme, b_bf[:, ca[g]], preferred_element_type=f32
            ).astype(bf16)
            zkb_ref[g] = jnp.dot(
                a_yme, b_bf[:, cb[g]], preferred_element_type=f32
            ).astype(bf16)

        p2 = {}
        for g in range(G):
            p1["a", g].wait()
            sua_ref[g] = (
                zka_ref[g, pl.ds(o_sa, mq), :].astype(f32)
                + r1a_ref[g, pl.ds(o_sa, mq), :].astype(f32)
            ).astype(bf16)
            p2["a", g] = exchange(
                sua_ref.at[g], r2a_ref.at[g], 1, 0, g, yn
            )
            p2["a", g].start()
            p1["b", g].wait()
            sub_ref[g] = (
                zkb_ref[g, pl.ds(o_sb, mq), :].astype(f32)
                + r1b_ref[g, pl.ds(o_sb, mq), :].astype(f32)
            ).astype(bf16)
            p2["b", g] = exchange(
                sub_ref.at[g], r2b_ref.at[g], 1, 1, g, xn
            )
            p2["b", g].start()

        ka, kb = {}, {}
        for g in range(G):
            ka[g] = (
                zka_ref[g, pl.ds(o_ka, mq), :].astype(f32)
                + r1a_ref[g, pl.ds(o_ka, mq), :].astype(f32)
            )
            kb[g] = (
                zkb_ref[g, pl.ds(o_kb, mq), :].astype(f32)
                + r1b_ref[g, pl.ds(o_kb, mq), :].astype(f32)
            )

        p3, p41 = {}, {}
        for g in range(G):
            p2["a", g].wait()
            wa = _gelu(ka[g] + r2a_ref[g].astype(f32))
            ga_ref[g] = wa.astype(bf16)
            p3["a", g] = exchange(ga_ref.at[g], r3a_ref.at[g], 2, 0, g, yn)
            p3["a", g].start()
            p41["a", g] = exchange(
                ga_ref.at[g], r4a_ref.at[g, 0], 3, 0, g, xn
            )
            p41["a", g].start()
            out_ref[pl.ds(rx_me + o_ka, mq), ca[g]] = wa

            p2["b", g].wait()
            wb = _gelu(kb[g] + r2b_ref[g].astype(f32))
            gb_ref[g] = wb.astype(bf16)
            p3["b", g] = exchange(gb_ref.at[g], r3b_ref.at[g], 2, 1, g, xn)
            p3["b", g].start()
            p41["b", g] = exchange(
                gb_ref.at[g], r4b_ref.at[g, 0], 3, 1, g, yn
            )
            p41["b", g].start()
            out_ref[pl.ds(ry_me + o_kb, mq), cb[g]] = wb

        p42 = {}
        for g in range(G):
            p3["a", g].wait()
            p42["a", g] = exchange(
                r3a_ref.at[g], r4a_ref.at[g, 1], 4, 0, g, xn
            )
            p42["a", g].start()
            out_ref[pl.ds(rx_me + o_sa, mq), ca[g]] = r3a_ref[g].astype(f32)

            p3["b", g].wait()
            p42["b", g] = exchange(
                r3b_ref.at[g], r4b_ref.at[g, 1], 4, 1, g, yn
            )
            p42["b", g].start()
            out_ref[pl.ds(ry_me + o_sb, mq), cb[g]] = r3b_ref[g].astype(f32)

            p41["a", g].wait()
            out_ref[pl.ds(rx_nb + o_ka, mq), ca[g]] = r4a_ref[g, 0].astype(f32)
            p41["b", g].wait()
            out_ref[pl.ds(ry_nb + o_kb, mq), cb[g]] = r4b_ref[g, 0].astype(f32)

        for g in range(G):
            p42["a", g].wait()
            out_ref[pl.ds(rx_nb + o_sa, mq), ca[g]] = r4a_ref[g, 1].astype(f32)
            p42["b", g].wait()
            out_ref[pl.ds(ry_nb + o_sb, mq), cb[g]] = r4b_ref[g, 1].astype(f32)

    return pl.pallas_call(
        body,
        out_shape=jax.ShapeDtypeStruct((m, n), f32),
        in_specs=[
            pl.BlockSpec(memory_space=pltpu.VMEM),
            pl.BlockSpec(memory_space=pltpu.VMEM),
        ],
        out_specs=pl.BlockSpec(memory_space=pltpu.VMEM),
        scratch_shapes=[
            pltpu.VMEM((G, mh, cw), bf16),
            pltpu.VMEM((G, mh, cw), bf16),
            pltpu.VMEM((G, mh, cw), bf16),
            pltpu.VMEM((G, mh, cw), bf16),
            pltpu.VMEM((G, mh, cw), bf16),
            pltpu.VMEM((G, mh, cw), bf16),
            pltpu.VMEM((G, mq, cw), bf16),
            pltpu.VMEM((G, mq, cw), bf16),
            pltpu.VMEM((G, mq, cw), bf16),
            pltpu.VMEM((G, mq, cw), bf16),
            pltpu.VMEM((G, mq, cw), bf16),
            pltpu.VMEM((G, mq, cw), bf16),
            pltpu.VMEM((G, mq, cw), bf16),
            pltpu.VMEM((G, mq, cw), bf16),
            pltpu.VMEM((G, 2, mq, cw), bf16),
            pltpu.VMEM((G, 2, mq, cw), bf16),
            pltpu.SemaphoreType.DMA((5, 2, G)),
            pltpu.SemaphoreType.DMA((5, 2, G)),
        ],
        compiler_params=pltpu.CompilerParams(collective_id=0),
    )(A, B)
